# baseline (speedup 1.0000x reference)
import jax
import jax.numpy as jnp
from jax import lax
from jax.experimental import pallas as pl
from jax.experimental.pallas import tpu as pltpu
from jax.experimental.pallas import tpu_sc as plsc

EMBED = 64
BATCH = 16384
L = 16
NC, NS = 2, 16
NW = NC * NS
BPW = BATCH // NW
NCH = 4
CH = BPW // NCH

PI = 3.14159265358979
TWO_PI = 6.28318530717959

S3 = -0.16664433
S5 = 0.008315025
S7 = -0.00019337327
S9 = 2.1777885e-06
C0 = 0.9999992
C2 = -0.49999422
C4 = 0.041659776
C6 = -0.001385879
C8 = 2.4202942e-05
C10 = -2.1972964e-07


def _sc_body(t_hbm, lab_hbm, table_hbm, inv_hbm, out_hbm,
             t_v, idx_v, rows_v, inv_v, sems):
    wid = lax.axis_index("s") * NC + lax.axis_index("c")
    base = wid * BPW
    pltpu.sync_copy(t_hbm.at[pl.ds(base, BPW)], t_v)
    pltpu.sync_copy(lab_hbm.at[pl.ds(wid * NCH, NCH), :], idx_v)
    pltpu.sync_copy(inv_hbm, inv_v)

    copies = [
        pltpu.async_copy(table_hbm.at[idx_v.at[j]],
                         rows_v.at[pl.ds(j * CH, CH), :],
                         sems.at[j])
        for j in range(NCH)
    ]

    inv0 = inv_v[pl.ds(0, L)]
    inv1 = inv_v[pl.ds(L, L)]

    def row_body(b, carry):
        grp = (b // L) * L
        lane = b - grp
        tg = t_v[pl.ds(grp, L)]
        tb = lax.gather(
            tg, lax.broadcast(lane, (L, 1)),
            lax.GatherDimensionNumbers(offset_dims=(),
                                       collapsed_slice_dims=(0,),
                                       start_index_map=(0,)),
            (1,), mode=lax.GatherScatterMode.PROMISE_IN_BOUNDS)
        a0 = tb * inv0
        a1 = tb * inv1
        r0 = lax.rem(a0 + PI, TWO_PI) - PI
        r1 = lax.rem(a1 + PI, TWO_PI) - PI
        z0 = r0 * r0
        z1 = r1 * r1
        p0 = ((S9 * z0 + S7) * z0 + S5) * z0 + S3
        p1 = ((S9 * z1 + S7) * z1 + S5) * z1 + S3
        sin0 = r0 + r0 * z0 * p0
        sin1 = r1 + r1 * z1 * p1
        cos0 = C0 + z0 * (C2 + z0 * (C4 + z0 * (C6 + z0 * (C8 + z0 * C10))))
        cos1 = C0 + z1 * (C2 + z1 * (C4 + z1 * (C6 + z1 * (C8 + z1 * C10))))
        for q, pos in enumerate((sin0, sin1, cos0, cos1)):
            x = rows_v[b, pl.ds(q * L, L)] + pos
            y = x / (1.0 + jnp.exp(-x))
            rows_v[b, pl.ds(q * L, L)] = y
        return carry

    for j in range(NCH):
        copies[j].wait()
        lax.fori_loop(j * CH, (j + 1) * CH, row_body, 0, unroll=2)

    pltpu.sync_copy(rows_v, out_hbm.at[pl.ds(base, BPW), :])


def kernel(t, label, table):
    label2 = label.astype(jnp.int32).reshape(BATCH // CH, CH)
    inv = (1.0 / (10000.0 ** (jnp.arange(1, EMBED, 2, dtype=jnp.float32)
                              / EMBED))).astype(jnp.float32)
    mesh = plsc.VectorSubcoreMesh(core_axis_name="c", subcore_axis_name="s")
    f = pl.kernel(
        _sc_body,
        mesh=mesh,
        compiler_params=pltpu.CompilerParams(use_tc_tiling_on_sc=False),
        out_type=jax.ShapeDtypeStruct((BATCH, EMBED), jnp.float32),
        scratch_types=[
            pltpu.VMEM((BPW,), jnp.float32),
            pltpu.VMEM((NCH, CH), jnp.int32),
            pltpu.VMEM((BPW, EMBED), jnp.float32),
            pltpu.VMEM((2 * L,), jnp.float32),
            pltpu.SemaphoreType.DMA((NCH,)),
        ],
    )
    return f(t, label2, table, inv)


# speedup vs baseline: 1.5969x; 1.5969x over previous
import jax
import jax.numpy as jnp
from jax import lax
from jax.experimental import pallas as pl
from jax.experimental.pallas import tpu as pltpu
from jax.experimental.pallas import tpu_sc as plsc

EMBED = 64
BATCH = 16384
L = 16
NC, NS = 2, 16
NW = NC * NS
BPW = BATCH // NW
NCH = 16
CH = BPW // NCH
GPC = CH // L

PI = 3.14159265358979
TWO_PI = 6.28318530717959

S3 = -0.16664433
S5 = 0.008315025
S7 = -0.00019337327
S9 = 2.1777885e-06
C0 = 0.9999992
C2 = -0.49999422
C4 = 0.041659776
C6 = -0.001385879
C8 = 2.4202942e-05
C10 = -2.1972964e-07


def _sc_body(t_hbm, lab_hbm, table_hbm, inv_hbm, out_hbm,
             t_v, lab_v, rows_v, out_v, inv_v, sems):
    wid = lax.axis_index("s") * NC + lax.axis_index("c")
    base = wid * BPW
    pltpu.sync_copy(t_hbm.at[pl.ds(base, BPW)], t_v)
    pltpu.sync_copy(lab_hbm.at[pl.ds(base, BPW)], lab_v)
    pltpu.sync_copy(inv_hbm, inv_v)

    inv0 = inv_v[pl.ds(0, L)]
    inv1 = inv_v[pl.ds(L, L)]

    def fire_chunk(j):
        j2 = j % 2

        def fg(g, c):
            vg = lab_v[pl.ds(j * CH + g * L, L)]
            for i in range(L):
                pltpu.async_copy(
                    table_hbm.at[pl.ds(vg[i], 1), :],
                    rows_v.at[j2, pl.ds(g * L + i, 1), :],
                    sems.at[j2])
            return c

        lax.fori_loop(0, GPC, fg, 0)

    def wait_chunk(j):
        pltpu.make_async_copy(table_hbm.at[pl.ds(0, CH), :],
                              rows_v.at[j % 2], sems.at[j % 2]).wait()

    fire_chunk(0)

    def chunk_body(j, carry):
        @pl.when(j + 1 < NCH)
        def _():
            fire_chunk(j + 1)

        wait_chunk(j)
        j2 = j % 2

        def group_body(g, c2):
            b0 = j * CH + g * L
            tg = t_v[pl.ds(b0, L)]
            for i in range(L):
                tb = lax.broadcast(tg[i], (L,))
                bl = g * L + i
                a0 = tb * inv0
                a1 = tb * inv1
                r0 = lax.rem(a0 + PI, TWO_PI) - PI
                r1 = lax.rem(a1 + PI, TWO_PI) - PI
                z0 = r0 * r0
                z1 = r1 * r1
                p0 = ((S9 * z0 + S7) * z0 + S5) * z0 + S3
                p1 = ((S9 * z1 + S7) * z1 + S5) * z1 + S3
                sin0 = r0 + r0 * z0 * p0
                sin1 = r1 + r1 * z1 * p1
                cos0 = C0 + z0 * (C2 + z0 * (C4 + z0 * (C6 + z0 * (C8 + z0 * C10))))
                cos1 = C0 + z1 * (C2 + z1 * (C4 + z1 * (C6 + z1 * (C8 + z1 * C10))))
                for q, pos in enumerate((sin0, sin1, cos0, cos1)):
                    x = rows_v[j2, bl, pl.ds(q * L, L)] + pos
                    y = x / (1.0 + jnp.exp(-x))
                    out_v[b0 + i, pl.ds(q * L, L)] = y
            return c2

        lax.fori_loop(0, GPC, group_body, 0)
        return carry

    lax.fori_loop(0, NCH, chunk_body, 0)
    pltpu.sync_copy(out_v, out_hbm.at[pl.ds(base, BPW), :])


def kernel(t, label, table):
    lab = label.astype(jnp.int32)
    inv = (1.0 / (10000.0 ** (jnp.arange(1, EMBED, 2, dtype=jnp.float32)
                              / EMBED))).astype(jnp.float32)
    mesh = plsc.VectorSubcoreMesh(core_axis_name="c", subcore_axis_name="s")
    f = pl.kernel(
        _sc_body,
        mesh=mesh,
        out_type=jax.ShapeDtypeStruct((BATCH, EMBED), jnp.float32),
        scratch_types=[
            pltpu.VMEM((BPW,), jnp.float32),
            pltpu.VMEM((BPW,), jnp.int32),
            pltpu.VMEM((2, CH, EMBED), jnp.float32),
            pltpu.VMEM((BPW, EMBED), jnp.float32),
            pltpu.VMEM((2 * L,), jnp.float32),
            pltpu.SemaphoreType.DMA((2,)),
        ],
    )
    return f(t, lab, table, inv)
